# baseline (speedup 1.0000x reference)
import functools

import jax
import jax.numpy as jnp
from jax import lax
from jax.experimental import pallas as pl
from jax.experimental.pallas import tpu as pltpu
from jax.experimental.pallas import tpu_sc as plsc

N = 50000
E = 800000
T = 16
H = 64
HH = H // 2
O = 10

NS = 16
NC = 2

CM = 128
ECH = E // CM
EMAX = ECH // NS + 1
EREM = ECH - NS * (ECH // NS)
DMAX = ECH // (NS * NC) + 1
DREM = ECH - NS * NC * (ECH // (NS * NC))

NBUF = 4
SB = 16
PFULL = 24
TMAX = 8

ZSPAN = 3200
WZR = 125
WSPAN = N // NS
WCH = WSPAN // WZR

_mesh = plsc.VectorSubcoreMesh(core_axis_name="c", subcore_axis_name="s")


@functools.partial(
    pl.kernel,
    mesh=_mesh,
    compiler_params=pltpu.CompilerParams(use_tc_tiling_on_sc=False),
    out_type=[jax.ShapeDtypeStruct((N,), jnp.float32),
              jax.ShapeDtypeStruct((N,), jnp.float32)],
    scratch_types=[
        pltpu.VMEM((DMAX, CM), jnp.int32),
        pltpu.VMEM((CM,), jnp.float32),
        pltpu.VMEM((ZSPAN,), jnp.float32),
        pltpu.VMEM_SHARED((N,), jnp.float32),
        pltpu.SemaphoreType.DMA,
    ],
)
def _deg_kernel(er_hbm, ones_hbm, zeros_hbm,
                deg0_hbm, deg1_hbm,
                dstbuf_v, ones_v, zstage_v, acc_sh, ssem):
    c = lax.axis_index("c")
    s = lax.axis_index("s")
    wid = s * NC + c
    cnt = jnp.where(wid < DREM, DMAX, DMAX - 1)

    pltpu.sync_copy(ones_hbm, ones_v)
    pltpu.sync_copy(zeros_hbm, zstage_v)

    @pl.when(wid < DREM)
    def _():
        pltpu.sync_copy(er_hbm.at[1, pl.ds(wid * DMAX, DMAX)], dstbuf_v)

    @pl.when(wid >= DREM)
    def _():
        b0 = DREM * DMAX + (wid - DREM) * (DMAX - 1)
        pltpu.sync_copy(er_hbm.at[1, pl.ds(b0, DMAX - 1)],
                        dstbuf_v.at[pl.ds(0, DMAX - 1)])

    @pl.when(s < NS - 1)
    def _():
        pltpu.sync_copy(zstage_v, acc_sh.at[pl.ds(s * ZSPAN, ZSPAN)])

    @pl.when(s == NS - 1)
    def _():
        rem = N - (NS - 1) * ZSPAN
        pltpu.sync_copy(zstage_v.at[pl.ds(0, rem)],
                        acc_sh.at[pl.ds((NS - 1) * ZSPAN, rem)])

    plsc.subcore_barrier()

    def body(j, carry):
        pltpu.async_copy(ones_v, acc_sh.at[dstbuf_v.at[j]], ssem, add=True)
        return carry

    lax.fori_loop(0, cnt, body, 0)

    def drain(j, carry):
        pltpu.make_async_copy(ones_v, acc_sh.at[pl.ds(0, CM)], ssem).wait()
        return carry

    lax.fori_loop(0, cnt, drain, 0)

    plsc.subcore_barrier()

    def wb(out_hbm):
        @pl.when(s < NS - 1)
        def _():
            r0 = s * ZSPAN
            pltpu.sync_copy(acc_sh.at[pl.ds(r0, ZSPAN)], zstage_v)
            pltpu.sync_copy(zstage_v, out_hbm.at[pl.ds(r0, ZSPAN)])

        @pl.when(s == NS - 1)
        def _():
            rem = N - (NS - 1) * ZSPAN
            r0 = (NS - 1) * ZSPAN
            pltpu.sync_copy(acc_sh.at[pl.ds(r0, rem)], zstage_v.at[pl.ds(0, rem)])
            pltpu.sync_copy(zstage_v.at[pl.ds(0, rem)], out_hbm.at[pl.ds(r0, rem)])

    @pl.when(c == 0)
    def _():
        wb(deg0_hbm)

    @pl.when(c == 1)
    def _():
        wb(deg1_hbm)


@functools.partial(
    pl.kernel,
    mesh=_mesh,
    compiler_params=pltpu.CompilerParams(use_tc_tiling_on_sc=False),
    out_type=[jax.ShapeDtypeStruct((N, HH), jnp.float32),
              jax.ShapeDtypeStruct((N, HH), jnp.float32)],
    scratch_types=(
        [pltpu.VMEM((SB, CM), jnp.int32) for _ in range(4)]
        + [pltpu.VMEM((TMAX - 1, CM), jnp.int32) for _ in range(2)]
        + [pltpu.VMEM((CM, HH), jnp.float32) for _ in range(NBUF)]
        + [pltpu.VMEM_SHARED((N, HH), jnp.float32)]
        + [pltpu.SemaphoreType.DMA for _ in range(2 + 2 * NBUF)]
    ),
)
def _edge_kernel(glo_hbm, ghi_hbm, er_hbm, zrows_hbm,
                 alo_hbm, ahi_hbm, *refs):
    sbuf = refs[0:2]
    dbuf = refs[2:4]
    tsrc, tdst = refs[4:6]
    rows = refs[6:6 + NBUF]
    acc_sh = refs[6 + NBUF]
    psem = refs[7 + NBUF:9 + NBUF]
    gsem = refs[9 + NBUF:9 + 2 * NBUF]
    ssem = refs[9 + 2 * NBUF:9 + 3 * NBUF]

    c = lax.axis_index("c")
    s = lax.axis_index("s")
    cnt = jnp.where(s < EREM, EMAX, EMAX - 1)
    base = s * (EMAX - 1) + jnp.minimum(s, EREM)

    @pl.when(s < EREM)
    def _():
        pltpu.sync_copy(er_hbm.at[0, pl.ds(base + PFULL * SB, TMAX - 1)], tsrc)
        pltpu.sync_copy(er_hbm.at[1, pl.ds(base + PFULL * SB, TMAX - 1)], tdst)

    @pl.when(s >= EREM)
    def _():
        pltpu.sync_copy(er_hbm.at[0, pl.ds(base + PFULL * SB, TMAX - 2)],
                        tsrc.at[pl.ds(0, TMAX - 2)])
        pltpu.sync_copy(er_hbm.at[1, pl.ds(base + PFULL * SB, TMAX - 2)],
                        tdst.at[pl.ds(0, TMAX - 2)])

    pltpu.sync_copy(er_hbm.at[0, pl.ds(base, SB)], sbuf[0])
    pltpu.sync_copy(er_hbm.at[1, pl.ds(base, SB)], dbuf[0])

    pltpu.sync_copy(zrows_hbm, rows[0])

    def zbody(k, carry):
        r = s * WSPAN + k * WZR
        pltpu.sync_copy(rows[0].at[pl.ds(0, WZR)], acc_sh.at[pl.ds(r, WZR)])
        return carry

    lax.fori_loop(0, WCH, zbody, 0)
    plsc.subcore_barrier()

    def run(g_hbm):
        def wait_scatter(b):
            pltpu.make_async_copy(rows[b], acc_sh.at[pl.ds(0, CM)],
                                  ssem[b]).wait()

        def wait_gather(b):
            pltpu.make_async_copy(g_hbm.at[pl.ds(0, CM)], rows[b],
                                  gsem[b]).wait()

        def step(k, carry):
            for pp in range(2):
                for t in range(SB):
                    p = k * 2 + pp
                    i = p * SB + t
                    b = t % NBUF
                    bj = (t - 1) % NBUF

                    @pl.when(i >= 1)
                    def _():
                        wait_gather(bj)
                        if t == 0:
                            idxrow = dbuf[(pp + 1) % 2].at[SB - 1]
                        else:
                            idxrow = dbuf[pp].at[t - 1]
                        pltpu.async_copy(rows[bj], acc_sh.at[idxrow],
                                         ssem[bj], add=True)

                    if t == 0:
                        @pl.when(p >= 1)
                        def _():
                            pltpu.make_async_copy(
                                er_hbm.at[0, pl.ds(base, SB)], sbuf[pp],
                                psem[pp]).wait()
                            pltpu.make_async_copy(
                                er_hbm.at[0, pl.ds(base, SB)], dbuf[pp],
                                psem[pp]).wait()

                    if t == 4:
                        nb = (pp + 1) % 2

                        @pl.when(p < PFULL - 1)
                        def _():
                            off = base + (p + 1) * SB
                            pltpu.async_copy(er_hbm.at[0, pl.ds(off, SB)],
                                             sbuf[nb], psem[nb])
                            pltpu.async_copy(er_hbm.at[1, pl.ds(off, SB)],
                                             dbuf[nb], psem[nb])

                    @pl.when(i >= NBUF)
                    def _():
                        wait_scatter(b)

                    pltpu.async_copy(g_hbm.at[sbuf[pp].at[t]], rows[b],
                                     gsem[b])

            return carry

        lax.fori_loop(0, PFULL // 2, step, 0)

        for t in range(TMAX):
            i = PFULL * SB + t
            b = t % NBUF
            bj = (t - 1) % NBUF

            @pl.when(i - 1 < cnt)
            def _():
                wait_gather(bj)
                if t == 0:
                    idxrow = dbuf[1].at[SB - 1]
                else:
                    idxrow = tdst.at[t - 1]
                pltpu.async_copy(rows[bj], acc_sh.at[idxrow], ssem[bj],
                                 add=True)

            if t < TMAX - 1:
                @pl.when(i < cnt)
                def _():
                    wait_scatter(b)
                    pltpu.async_copy(g_hbm.at[tsrc.at[t]], rows[b], gsem[b])

        for b in range(NBUF):
            wait_scatter(b)

    @pl.when(c == 0)
    def _():
        run(glo_hbm)

    @pl.when(c == 1)
    def _():
        run(ghi_hbm)

    plsc.subcore_barrier()

    def wb(out_hbm):
        def wbody(k, carry):
            r = s * WSPAN + k * WZR
            b = rows[0]
            pltpu.sync_copy(acc_sh.at[pl.ds(r, WZR)], b.at[pl.ds(0, WZR)])
            pltpu.sync_copy(b.at[pl.ds(0, WZR)], out_hbm.at[pl.ds(r, WZR)])
            return carry

        lax.fori_loop(0, WCH, wbody, 0)

    @pl.when(c == 0)
    def _():
        wb(alo_hbm)

    @pl.when(c == 1)
    def _():
        wb(ahi_hbm)


NP = N // 4
RB = NP
GRIDP = 1


def _dinv_packed(d0_ref, d1_ref, dbc_ref):
    deg = d0_ref[...] + d1_ref[...] + 1.0
    dinv4 = lax.rsqrt(deg)
    return jnp.dot(dinv4, dbc_ref[...], preferred_element_type=jnp.float32)


def _embed_body(x_ref, d0_ref, d1_ref, xrep_ref, iot_ref, w1lo_ref, w1hi_ref,
                dbc_ref, glo_ref, ghi_ref):
    xf = x_ref[...].astype(jnp.float32)
    xr = jnp.dot(xf, xrep_ref[...], preferred_element_type=jnp.float32)
    oh = (xr == iot_ref[...]).astype(jnp.float32)
    dp = _dinv_packed(d0_ref, d1_ref, dbc_ref)
    glo_ref[...] = jnp.dot(oh, w1lo_ref[...],
                           preferred_element_type=jnp.float32) * dp
    ghi_ref[...] = jnp.dot(oh, w1hi_ref[...],
                           preferred_element_type=jnp.float32) * dp


def _h_body(alo_ref, ahi_ref, glo_ref, ghi_ref, d0_ref, d1_ref,
            blo_ref, bhi_ref, dbc_ref, hlo_ref, hhi_ref):
    dp = _dinv_packed(d0_ref, d1_ref, dbc_ref)
    hlo_ref[...] = jnp.maximum(
        dp * (alo_ref[...] + glo_ref[...]) + blo_ref[...], 0.0)
    hhi_ref[...] = jnp.maximum(
        dp * (ahi_ref[...] + ghi_ref[...]) + bhi_ref[...], 0.0)


def _g_body(hlo_ref, hhi_ref, d0_ref, d1_ref,
            waa_ref, wba_ref, wab_ref, wbb_ref, dbc_ref, olo_ref, ohi_ref):
    dp = _dinv_packed(d0_ref, d1_ref, dbc_ref)
    hlo = hlo_ref[...]
    hhi = hhi_ref[...]
    olo_ref[...] = (jnp.dot(hlo, waa_ref[...],
                            preferred_element_type=jnp.float32)
                    + jnp.dot(hhi, wba_ref[...],
                              preferred_element_type=jnp.float32)) * dp
    ohi_ref[...] = (jnp.dot(hlo, wab_ref[...],
                            preferred_element_type=jnp.float32)
                    + jnp.dot(hhi, wbb_ref[...],
                              preferred_element_type=jnp.float32)) * dp


def _out_body(alo_ref, ahi_ref, glo_ref, ghi_ref, d0_ref, d1_ref,
              b2lo_ref, b2hi_ref, klo_ref, khi_ref, bpt_ref, dbc_ref, o_ref):
    dp = _dinv_packed(d0_ref, d1_ref, dbc_ref)
    hlo = jnp.maximum(dp * (alo_ref[...] + glo_ref[...]) + b2lo_ref[...], 0.0)
    hhi = jnp.maximum(dp * (ahi_ref[...] + ghi_ref[...]) + b2hi_ref[...], 0.0)
    o_ref[...] = (jnp.dot(hlo, klo_ref[...],
                          preferred_element_type=jnp.float32)
                  + jnp.dot(hhi, khi_ref[...],
                            preferred_element_type=jnp.float32)
                  + bpt_ref[...])


def _prow_spec(w):
    return pl.BlockSpec((RB, w), lambda i: (i, 0))


def _full_spec(a, b):
    return pl.BlockSpec((a, b), lambda i: (0, 0))


_P128 = jax.ShapeDtypeStruct((NP, 128), jnp.float32)

_embed_call = pl.pallas_call(
    _embed_body,
    grid=(GRIDP,),
    in_specs=[_prow_spec(4), _prow_spec(4), _prow_spec(4),
              _full_spec(4, 4 * T), _full_spec(1, 4 * T),
              _full_spec(4 * T, 128), _full_spec(4 * T, 128),
              _full_spec(4, 128)],
    out_specs=[_prow_spec(128), _prow_spec(128)],
    out_shape=[_P128, _P128],
)

_h_call = pl.pallas_call(
    _h_body,
    grid=(GRIDP,),
    in_specs=[_prow_spec(128), _prow_spec(128), _prow_spec(128),
              _prow_spec(128), _prow_spec(4), _prow_spec(4),
              _full_spec(1, 128), _full_spec(1, 128), _full_spec(4, 128)],
    out_specs=[_prow_spec(128), _prow_spec(128)],
    out_shape=[_P128, _P128],
)

_g_call = pl.pallas_call(
    _g_body,
    grid=(GRIDP,),
    in_specs=[_prow_spec(128), _prow_spec(128), _prow_spec(4), _prow_spec(4),
              _full_spec(128, 128), _full_spec(128, 128),
              _full_spec(128, 128), _full_spec(128, 128),
              _full_spec(4, 128)],
    out_specs=[_prow_spec(128), _prow_spec(128)],
    out_shape=[_P128, _P128],
)

_out_call = pl.pallas_call(
    _out_body,
    grid=(GRIDP,),
    in_specs=[_prow_spec(128), _prow_spec(128), _prow_spec(128),
              _prow_spec(128), _prow_spec(4), _prow_spec(4),
              _full_spec(1, 128), _full_spec(1, 128),
              _full_spec(128, 4 * O), _full_spec(128, 4 * O),
              _full_spec(1, 4 * O), _full_spec(4, 128)],
    out_specs=_prow_spec(4 * O),
    out_shape=jax.ShapeDtypeStruct((NP, 4 * O), jnp.float32),
)


def _kron4(a):
    return jnp.kron(jnp.eye(4, dtype=jnp.float32), a.astype(jnp.float32))


def kernel(x, edge_index, W1, b1, W2, b2, Wp, bp):
    er = edge_index.reshape(2, ECH, CM)

    ones_m = jnp.ones((CM,), jnp.float32)
    zeros_z = jnp.zeros((ZSPAN,), jnp.float32)
    zrows = jnp.zeros((CM, HH), jnp.float32)

    xrep = _kron4(jnp.ones((1, T), jnp.float32))
    iot = jnp.tile(jnp.arange(T, dtype=jnp.float32), 4).reshape(1, 4 * T)
    w1lo = _kron4(W1[:, :HH])
    w1hi = _kron4(W1[:, HH:])
    dbc = _kron4(jnp.ones((1, HH), jnp.float32))
    b1lo = jnp.tile(b1[:HH], 4).reshape(1, 128)
    b1hi = jnp.tile(b1[HH:], 4).reshape(1, 128)
    b2lo = jnp.tile(b2[:HH], 4).reshape(1, 128)
    b2hi = jnp.tile(b2[HH:], 4).reshape(1, 128)
    waa = _kron4(W2[:HH, :HH])
    wba = _kron4(W2[HH:, :HH])
    wab = _kron4(W2[:HH, HH:])
    wbb = _kron4(W2[HH:, HH:])
    klo = _kron4(Wp[:HH, :])
    khi = _kron4(Wp[HH:, :])
    bpt = jnp.tile(bp, 4).reshape(1, 4 * O)

    xp4 = x.reshape(NP, 4)

    deg0, deg1 = _deg_kernel(er, ones_m, zeros_z)
    d04 = deg0.reshape(NP, 4)
    d14 = deg1.reshape(NP, 4)

    g1lo_p, g1hi_p = _embed_call(xp4, d04, d14, xrep, iot, w1lo, w1hi, dbc)

    alo1, ahi1 = _edge_kernel(g1lo_p.reshape(N, HH), g1hi_p.reshape(N, HH),
                              er, zrows)

    h1lo_p, h1hi_p = _h_call(alo1.reshape(NP, 128), ahi1.reshape(NP, 128),
                             g1lo_p, g1hi_p, d04, d14, b1lo, b1hi, dbc)
    g2lo_p, g2hi_p = _g_call(h1lo_p, h1hi_p, d04, d14,
                             waa, wba, wab, wbb, dbc)

    alo2, ahi2 = _edge_kernel(g2lo_p.reshape(N, HH), g2hi_p.reshape(N, HH),
                              er, zrows)

    out_p = _out_call(alo2.reshape(NP, 128), ahi2.reshape(NP, 128),
                      g2lo_p, g2hi_p, d04, d14, b2lo, b2hi,
                      klo, khi, bpt, dbc)
    return out_p.reshape(N, O)


# speedup vs baseline: 1.0087x; 1.0087x over previous
import functools

import jax
import jax.numpy as jnp
from jax import lax
from jax.experimental import pallas as pl
from jax.experimental.pallas import tpu as pltpu
from jax.experimental.pallas import tpu_sc as plsc

N = 50000
E = 800000
T = 16
H = 64
HH = H // 2
O = 10

NS = 16
NC = 2

CM = 128
ECH = E // CM
EMAX = ECH // NS + 1
EREM = ECH - NS * (ECH // NS)
DMAX = ECH // (NS * NC) + 1
DREM = ECH - NS * NC * (ECH // (NS * NC))

NBUF = 4
SB = 16
PFULL = 24
TMAX = 8

ZSPAN = 3200
WZR = 125
WSPAN = N // NS
WCH = WSPAN // WZR

_mesh = plsc.VectorSubcoreMesh(core_axis_name="c", subcore_axis_name="s")


@functools.partial(
    pl.kernel,
    mesh=_mesh,
    compiler_params=pltpu.CompilerParams(use_tc_tiling_on_sc=False),
    out_type=[jax.ShapeDtypeStruct((N,), jnp.float32),
              jax.ShapeDtypeStruct((N,), jnp.float32)],
    scratch_types=[
        pltpu.VMEM((DMAX, CM), jnp.int32),
        pltpu.VMEM((CM,), jnp.float32),
        pltpu.VMEM((ZSPAN,), jnp.float32),
        pltpu.VMEM_SHARED((N,), jnp.float32),
        pltpu.SemaphoreType.DMA,
    ],
)
def _deg_kernel(er_hbm, ones_hbm, zeros_hbm,
                deg0_hbm, deg1_hbm,
                dstbuf_v, ones_v, zstage_v, acc_sh, ssem):
    c = lax.axis_index("c")
    s = lax.axis_index("s")
    wid = s * NC + c
    cnt = jnp.where(wid < DREM, DMAX, DMAX - 1)

    pltpu.sync_copy(ones_hbm, ones_v)
    pltpu.sync_copy(zeros_hbm, zstage_v)

    @pl.when(wid < DREM)
    def _():
        pltpu.sync_copy(er_hbm.at[1, pl.ds(wid * DMAX, DMAX)], dstbuf_v)

    @pl.when(wid >= DREM)
    def _():
        b0 = DREM * DMAX + (wid - DREM) * (DMAX - 1)
        pltpu.sync_copy(er_hbm.at[1, pl.ds(b0, DMAX - 1)],
                        dstbuf_v.at[pl.ds(0, DMAX - 1)])

    @pl.when(s < NS - 1)
    def _():
        pltpu.sync_copy(zstage_v, acc_sh.at[pl.ds(s * ZSPAN, ZSPAN)])

    @pl.when(s == NS - 1)
    def _():
        rem = N - (NS - 1) * ZSPAN
        pltpu.sync_copy(zstage_v.at[pl.ds(0, rem)],
                        acc_sh.at[pl.ds((NS - 1) * ZSPAN, rem)])

    plsc.subcore_barrier()

    def body(j, carry):
        pltpu.async_copy(ones_v, acc_sh.at[dstbuf_v.at[j]], ssem, add=True)
        return carry

    lax.fori_loop(0, cnt, body, 0)

    def drain(j, carry):
        pltpu.make_async_copy(ones_v, acc_sh.at[pl.ds(0, CM)], ssem).wait()
        return carry

    lax.fori_loop(0, cnt, drain, 0)

    plsc.subcore_barrier()

    def wb(out_hbm):
        @pl.when(s < NS - 1)
        def _():
            r0 = s * ZSPAN
            pltpu.sync_copy(acc_sh.at[pl.ds(r0, ZSPAN)], zstage_v)
            pltpu.sync_copy(zstage_v, out_hbm.at[pl.ds(r0, ZSPAN)])

        @pl.when(s == NS - 1)
        def _():
            rem = N - (NS - 1) * ZSPAN
            r0 = (NS - 1) * ZSPAN
            pltpu.sync_copy(acc_sh.at[pl.ds(r0, rem)], zstage_v.at[pl.ds(0, rem)])
            pltpu.sync_copy(zstage_v.at[pl.ds(0, rem)], out_hbm.at[pl.ds(r0, rem)])

    @pl.when(c == 0)
    def _():
        wb(deg0_hbm)

    @pl.when(c == 1)
    def _():
        wb(deg1_hbm)


@functools.partial(
    pl.kernel,
    mesh=_mesh,
    compiler_params=pltpu.CompilerParams(use_tc_tiling_on_sc=False),
    out_type=[jax.ShapeDtypeStruct((N, HH), jnp.float32),
              jax.ShapeDtypeStruct((N, HH), jnp.float32)],
    scratch_types=(
        [pltpu.VMEM((SB, CM), jnp.int32) for _ in range(4)]
        + [pltpu.VMEM((TMAX - 1, CM), jnp.int32) for _ in range(2)]
        + [pltpu.VMEM((CM, HH), jnp.float32) for _ in range(NBUF)]
        + [pltpu.VMEM_SHARED((N, HH), jnp.float32)]
        + [pltpu.SemaphoreType.DMA for _ in range(2 + 2 * NBUF)]
    ),
)
def _edge_kernel(glo_hbm, ghi_hbm, er_hbm, zrows_hbm,
                 alo_hbm, ahi_hbm, *refs):
    sbuf = refs[0:2]
    dbuf = refs[2:4]
    tsrc, tdst = refs[4:6]
    rows = refs[6:6 + NBUF]
    acc_sh = refs[6 + NBUF]
    psem = refs[7 + NBUF:9 + NBUF]
    gsem = refs[9 + NBUF:9 + 2 * NBUF]
    ssem = refs[9 + 2 * NBUF:9 + 3 * NBUF]

    c = lax.axis_index("c")
    s = lax.axis_index("s")
    cnt = jnp.where(s < EREM, EMAX, EMAX - 1)
    base = s * (EMAX - 1) + jnp.minimum(s, EREM)

    @pl.when(s < EREM)
    def _():
        pltpu.sync_copy(er_hbm.at[0, pl.ds(base + PFULL * SB, TMAX - 1)], tsrc)
        pltpu.sync_copy(er_hbm.at[1, pl.ds(base + PFULL * SB, TMAX - 1)], tdst)

    @pl.when(s >= EREM)
    def _():
        pltpu.sync_copy(er_hbm.at[0, pl.ds(base + PFULL * SB, TMAX - 2)],
                        tsrc.at[pl.ds(0, TMAX - 2)])
        pltpu.sync_copy(er_hbm.at[1, pl.ds(base + PFULL * SB, TMAX - 2)],
                        tdst.at[pl.ds(0, TMAX - 2)])

    pltpu.sync_copy(er_hbm.at[0, pl.ds(base, SB)], sbuf[0])
    pltpu.sync_copy(er_hbm.at[1, pl.ds(base, SB)], dbuf[0])

    pltpu.sync_copy(zrows_hbm, rows[0])

    def zbody(k, carry):
        r = s * WSPAN + k * WZR
        pltpu.async_copy(rows[0].at[pl.ds(0, WZR)], acc_sh.at[pl.ds(r, WZR)],
                         psem[0])
        return carry

    lax.fori_loop(0, WCH, zbody, 0)

    def zdrain(k, carry):
        pltpu.make_async_copy(rows[0].at[pl.ds(0, WZR)],
                              acc_sh.at[pl.ds(0, WZR)], psem[0]).wait()
        return carry

    lax.fori_loop(0, WCH, zdrain, 0)
    plsc.subcore_barrier()

    def run(g_hbm):
        def wait_scatter(b):
            pltpu.make_async_copy(rows[b], acc_sh.at[pl.ds(0, CM)],
                                  ssem[b]).wait()

        def wait_gather(b):
            pltpu.make_async_copy(g_hbm.at[pl.ds(0, CM)], rows[b],
                                  gsem[b]).wait()

        def step(k, carry):
            for pp in range(2):
                for t in range(SB):
                    p = k * 2 + pp
                    i = p * SB + t
                    b = t % NBUF
                    bj = (t - 1) % NBUF

                    @pl.when(i >= 1)
                    def _():
                        wait_gather(bj)
                        if t == 0:
                            idxrow = dbuf[(pp + 1) % 2].at[SB - 1]
                        else:
                            idxrow = dbuf[pp].at[t - 1]
                        pltpu.async_copy(rows[bj], acc_sh.at[idxrow],
                                         ssem[bj], add=True)

                    if t == 0:
                        @pl.when(p >= 1)
                        def _():
                            pltpu.make_async_copy(
                                er_hbm.at[0, pl.ds(base, SB)], sbuf[pp],
                                psem[pp]).wait()
                            pltpu.make_async_copy(
                                er_hbm.at[0, pl.ds(base, SB)], dbuf[pp],
                                psem[pp]).wait()

                    if t == 4:
                        nb = (pp + 1) % 2

                        @pl.when(p < PFULL - 1)
                        def _():
                            off = base + (p + 1) * SB
                            pltpu.async_copy(er_hbm.at[0, pl.ds(off, SB)],
                                             sbuf[nb], psem[nb])
                            pltpu.async_copy(er_hbm.at[1, pl.ds(off, SB)],
                                             dbuf[nb], psem[nb])

                    @pl.when(i >= NBUF)
                    def _():
                        wait_scatter(b)

                    pltpu.async_copy(g_hbm.at[sbuf[pp].at[t]], rows[b],
                                     gsem[b])

            return carry

        lax.fori_loop(0, PFULL // 2, step, 0)

        for t in range(TMAX):
            i = PFULL * SB + t
            b = t % NBUF
            bj = (t - 1) % NBUF

            @pl.when(i - 1 < cnt)
            def _():
                wait_gather(bj)
                if t == 0:
                    idxrow = dbuf[1].at[SB - 1]
                else:
                    idxrow = tdst.at[t - 1]
                pltpu.async_copy(rows[bj], acc_sh.at[idxrow], ssem[bj],
                                 add=True)

            if t < TMAX - 1:
                @pl.when(i < cnt)
                def _():
                    wait_scatter(b)
                    pltpu.async_copy(g_hbm.at[tsrc.at[t]], rows[b], gsem[b])

        for b in range(NBUF):
            wait_scatter(b)

    @pl.when(c == 0)
    def _():
        run(glo_hbm)

    @pl.when(c == 1)
    def _():
        run(ghi_hbm)

    plsc.subcore_barrier()

    def wb(out_hbm):
        def wbody(k, carry):
            r = s * WSPAN + k * WZR
            pltpu.async_copy(acc_sh.at[pl.ds(r, WZR)],
                             out_hbm.at[pl.ds(r, WZR)], psem[1])
            return carry

        lax.fori_loop(0, WCH, wbody, 0)

        def wdrain(k, carry):
            pltpu.make_async_copy(acc_sh.at[pl.ds(0, WZR)],
                                  out_hbm.at[pl.ds(0, WZR)], psem[1]).wait()
            return carry

        lax.fori_loop(0, WCH, wdrain, 0)

    @pl.when(c == 0)
    def _():
        wb(alo_hbm)

    @pl.when(c == 1)
    def _():
        wb(ahi_hbm)


NP = N // 4
RB = NP
GRIDP = 1


def _dinv_packed(d0_ref, d1_ref, dbc_ref):
    deg = d0_ref[...] + d1_ref[...] + 1.0
    dinv4 = lax.rsqrt(deg)
    return jnp.dot(dinv4, dbc_ref[...], preferred_element_type=jnp.float32)


def _embed_body(x_ref, d0_ref, d1_ref, xrep_ref, iot_ref, w1lo_ref, w1hi_ref,
                dbc_ref, glo_ref, ghi_ref):
    xf = x_ref[...].astype(jnp.float32)
    xr = jnp.dot(xf, xrep_ref[...], preferred_element_type=jnp.float32)
    oh = (xr == iot_ref[...]).astype(jnp.float32)
    dp = _dinv_packed(d0_ref, d1_ref, dbc_ref)
    glo_ref[...] = jnp.dot(oh, w1lo_ref[...],
                           preferred_element_type=jnp.float32) * dp
    ghi_ref[...] = jnp.dot(oh, w1hi_ref[...],
                           preferred_element_type=jnp.float32) * dp


def _h_body(alo_ref, ahi_ref, glo_ref, ghi_ref, d0_ref, d1_ref,
            blo_ref, bhi_ref, dbc_ref, hlo_ref, hhi_ref):
    dp = _dinv_packed(d0_ref, d1_ref, dbc_ref)
    hlo_ref[...] = jnp.maximum(
        dp * (alo_ref[...] + glo_ref[...]) + blo_ref[...], 0.0)
    hhi_ref[...] = jnp.maximum(
        dp * (ahi_ref[...] + ghi_ref[...]) + bhi_ref[...], 0.0)


def _g_body(hlo_ref, hhi_ref, d0_ref, d1_ref,
            waa_ref, wba_ref, wab_ref, wbb_ref, dbc_ref, olo_ref, ohi_ref):
    dp = _dinv_packed(d0_ref, d1_ref, dbc_ref)
    hlo = hlo_ref[...]
    hhi = hhi_ref[...]
    olo_ref[...] = (jnp.dot(hlo, waa_ref[...],
                            preferred_element_type=jnp.float32)
                    + jnp.dot(hhi, wba_ref[...],
                              preferred_element_type=jnp.float32)) * dp
    ohi_ref[...] = (jnp.dot(hlo, wab_ref[...],
                            preferred_element_type=jnp.float32)
                    + jnp.dot(hhi, wbb_ref[...],
                              preferred_element_type=jnp.float32)) * dp


def _out_body(alo_ref, ahi_ref, glo_ref, ghi_ref, d0_ref, d1_ref,
              b2lo_ref, b2hi_ref, klo_ref, khi_ref, bpt_ref, dbc_ref, o_ref):
    dp = _dinv_packed(d0_ref, d1_ref, dbc_ref)
    hlo = jnp.maximum(dp * (alo_ref[...] + glo_ref[...]) + b2lo_ref[...], 0.0)
    hhi = jnp.maximum(dp * (ahi_ref[...] + ghi_ref[...]) + b2hi_ref[...], 0.0)
    o_ref[...] = (jnp.dot(hlo, klo_ref[...],
                          preferred_element_type=jnp.float32)
                  + jnp.dot(hhi, khi_ref[...],
                            preferred_element_type=jnp.float32)
                  + bpt_ref[...])


def _prow_spec(w):
    return pl.BlockSpec((RB, w), lambda i: (i, 0))


def _full_spec(a, b):
    return pl.BlockSpec((a, b), lambda i: (0, 0))


_P128 = jax.ShapeDtypeStruct((NP, 128), jnp.float32)

_embed_call = pl.pallas_call(
    _embed_body,
    grid=(GRIDP,),
    in_specs=[_prow_spec(4), _prow_spec(4), _prow_spec(4),
              _full_spec(4, 4 * T), _full_spec(1, 4 * T),
              _full_spec(4 * T, 128), _full_spec(4 * T, 128),
              _full_spec(4, 128)],
    out_specs=[_prow_spec(128), _prow_spec(128)],
    out_shape=[_P128, _P128],
)

_h_call = pl.pallas_call(
    _h_body,
    grid=(GRIDP,),
    in_specs=[_prow_spec(128), _prow_spec(128), _prow_spec(128),
              _prow_spec(128), _prow_spec(4), _prow_spec(4),
              _full_spec(1, 128), _full_spec(1, 128), _full_spec(4, 128)],
    out_specs=[_prow_spec(128), _prow_spec(128)],
    out_shape=[_P128, _P128],
)

_g_call = pl.pallas_call(
    _g_body,
    grid=(GRIDP,),
    in_specs=[_prow_spec(128), _prow_spec(128), _prow_spec(4), _prow_spec(4),
              _full_spec(128, 128), _full_spec(128, 128),
              _full_spec(128, 128), _full_spec(128, 128),
              _full_spec(4, 128)],
    out_specs=[_prow_spec(128), _prow_spec(128)],
    out_shape=[_P128, _P128],
)

_out_call = pl.pallas_call(
    _out_body,
    grid=(GRIDP,),
    in_specs=[_prow_spec(128), _prow_spec(128), _prow_spec(128),
              _prow_spec(128), _prow_spec(4), _prow_spec(4),
              _full_spec(1, 128), _full_spec(1, 128),
              _full_spec(128, 4 * O), _full_spec(128, 4 * O),
              _full_spec(1, 4 * O), _full_spec(4, 128)],
    out_specs=_prow_spec(4 * O),
    out_shape=jax.ShapeDtypeStruct((NP, 4 * O), jnp.float32),
)


def _kron4(a):
    return jnp.kron(jnp.eye(4, dtype=jnp.float32), a.astype(jnp.float32))


def kernel(x, edge_index, W1, b1, W2, b2, Wp, bp):
    er = edge_index.reshape(2, ECH, CM)

    ones_m = jnp.ones((CM,), jnp.float32)
    zeros_z = jnp.zeros((ZSPAN,), jnp.float32)
    zrows = jnp.zeros((CM, HH), jnp.float32)

    xrep = _kron4(jnp.ones((1, T), jnp.float32))
    iot = jnp.tile(jnp.arange(T, dtype=jnp.float32), 4).reshape(1, 4 * T)
    w1lo = _kron4(W1[:, :HH])
    w1hi = _kron4(W1[:, HH:])
    dbc = _kron4(jnp.ones((1, HH), jnp.float32))
    b1lo = jnp.tile(b1[:HH], 4).reshape(1, 128)
    b1hi = jnp.tile(b1[HH:], 4).reshape(1, 128)
    b2lo = jnp.tile(b2[:HH], 4).reshape(1, 128)
    b2hi = jnp.tile(b2[HH:], 4).reshape(1, 128)
    waa = _kron4(W2[:HH, :HH])
    wba = _kron4(W2[HH:, :HH])
    wab = _kron4(W2[:HH, HH:])
    wbb = _kron4(W2[HH:, HH:])
    klo = _kron4(Wp[:HH, :])
    khi = _kron4(Wp[HH:, :])
    bpt = jnp.tile(bp, 4).reshape(1, 4 * O)

    xp4 = x.reshape(NP, 4)

    deg0, deg1 = _deg_kernel(er, ones_m, zeros_z)
    d04 = deg0.reshape(NP, 4)
    d14 = deg1.reshape(NP, 4)

    g1lo_p, g1hi_p = _embed_call(xp4, d04, d14, xrep, iot, w1lo, w1hi, dbc)

    alo1, ahi1 = _edge_kernel(g1lo_p.reshape(N, HH), g1hi_p.reshape(N, HH),
                              er, zrows)

    h1lo_p, h1hi_p = _h_call(alo1.reshape(NP, 128), ahi1.reshape(NP, 128),
                             g1lo_p, g1hi_p, d04, d14, b1lo, b1hi, dbc)
    g2lo_p, g2hi_p = _g_call(h1lo_p, h1hi_p, d04, d14,
                             waa, wba, wab, wbb, dbc)

    alo2, ahi2 = _edge_kernel(g2lo_p.reshape(N, HH), g2hi_p.reshape(N, HH),
                              er, zrows)

    out_p = _out_call(alo2.reshape(NP, 128), ahi2.reshape(NP, 128),
                      g2lo_p, g2hi_p, d04, d14, b2lo, b2hi,
                      klo, khi, bpt, dbc)
    return out_p.reshape(N, O)
